# Initial kernel scaffold; baseline (speedup 1.0000x reference)
import jax
import jax.numpy as jnp
from jax.experimental import pallas as pl


def kernel(x, edge_index, edge_attr, W_rel1, b_rel1, W_root1, a1, W_rel2, b_rel2, W_root2, a2, W_fc, b_fc):
    raise NotImplementedError("write your pallas kernel here")


# speedup vs baseline: 3.4371x; 3.4371x over previous
import functools

import jax
import jax.numpy as jnp
from jax import lax
from jax.experimental import pallas as pl
from jax.experimental.pallas import tpu as pltpu
from jax.experimental.pallas import tpu_sc as plsc

N = 10000
NPAD = 10240
D = 128
CHUNK = 80
NC, NS = 2, 16
BATCH = 10


def _make_segsum(E, mode):
    R = E // CHUNK
    rows_pw = R // (NC * NS) if mode == 1 else R // NS
    G = 5
    ngroups = rows_pw // G
    rows_per_tile = NPAD // NS
    mesh = plsc.VectorSubcoreMesh(core_axis_name="c", subcore_axis_name="s")

    @functools.partial(
        pl.kernel,
        mesh=mesh,
        out_type=jax.ShapeDtypeStruct((NC, NPAD, D), jnp.float32),
        scratch_types=[
            pltpu.VMEM((G, CHUNK), jnp.int32),
            pltpu.VMEM((G, CHUNK), jnp.int32),
            pltpu.VMEM((G * CHUNK // 8, D), jnp.float32),
            pltpu.VMEM((CHUNK, D), jnp.float32),
            pltpu.VMEM_SHARED((NPAD, D), jnp.float32),
            pltpu.SemaphoreType.DMA,
        ],
    )
    def seg(table_h, srcs_h, dst_h, wrep_h, out_h,
            src_v, dst_v, wrep_v, rows_v, agg_sh, sem):
        c = lax.axis_index("c")
        s = lax.axis_index("s")
        wid = s * NC + c if mode == 1 else s

        zvec = jnp.zeros((16,), jnp.float32)

        def zrow(r, _):
            for k in range(D // 16):
                rows_v[r, pl.ds(k * 16, 16)] = zvec
            return 0

        lax.fori_loop(0, CHUNK, zrow, 0)

        def zcopy(i, _):
            pltpu.sync_copy(
                rows_v, agg_sh.at[pl.ds(s * rows_per_tile + i * CHUNK, CHUNK)])
            return 0

        lax.fori_loop(0, rows_per_tile // CHUNK, zcopy, 0)
        plsc.subcore_barrier()

        def group_body(g, _):
            pltpu.sync_copy(srcs_h.at[c, wid, g], src_v)
            pltpu.sync_copy(dst_h.at[wid, g], dst_v)
            pltpu.sync_copy(wrep_h.at[wid, g], wrep_v)

            def chunk_body(j, _):
                pltpu.async_copy(table_h.at[src_v.at[j]], rows_v, sem).wait()

                def scale(e, _):
                    r = j * (CHUNK // 8) + e // 8
                    wsp = wrep_v[r, pl.ds((e % 8) * 16, 16)]
                    for k in range(D // 16):
                        rows_v[e, pl.ds(k * 16, 16)] = (
                            rows_v[e, pl.ds(k * 16, 16)] * wsp)
                    return 0

                lax.fori_loop(0, CHUNK, scale, 0)
                pltpu.sync_copy(rows_v, agg_sh.at[dst_v.at[j]], add=True)
                return 0

            lax.fori_loop(0, G, chunk_body, 0)
            return 0

        lax.fori_loop(0, ngroups, group_body, 0)
        plsc.subcore_barrier()

        pltpu.sync_copy(
            agg_sh.at[pl.ds(s * rows_per_tile, rows_per_tile)],
            out_h.at[c, pl.ds(s * rows_per_tile, rows_per_tile)])

    return seg


def _l1_body(aggp_ref, x_ref, wr_ref, wro_ref, b_ref, a_ref, out_ref):
    agg = aggp_ref[0] + aggp_ref[1]
    t = jnp.dot(agg, wr_ref[...], preferred_element_type=jnp.float32)
    t += jnp.dot(x_ref[...], wro_ref[...], preferred_element_type=jnp.float32)
    t += b_ref[...]
    h = jnp.where(t >= 0.0, t, a_ref[...] * t)
    out_ref[0] = h[:, :D]
    out_ref[1] = h[:, D:]


def _l2_body(agg_ref, h1_ref, wr_ref, wro_ref, b_ref, a_ref, wfc_ref,
             bfc_ref, out_ref):
    t = jnp.dot(agg_ref[0], wr_ref[:D], preferred_element_type=jnp.float32)
    t += jnp.dot(agg_ref[1], wr_ref[D:], preferred_element_type=jnp.float32)
    t += jnp.dot(h1_ref[0], wro_ref[:D], preferred_element_type=jnp.float32)
    t += jnp.dot(h1_ref[1], wro_ref[D:], preferred_element_type=jnp.float32)
    t += b_ref[...]
    h = jnp.where(t >= 0.0, t, a_ref[...] * t)
    out_ref[...] = (
        jnp.dot(h, wfc_ref[...], preferred_element_type=jnp.float32)
        + bfc_ref[...])


_RB = 1000


def _tc_layer1(aggp, x, W_rel1, b_rel1, W_root1, a1):
    grid = (N // _RB,)
    return pl.pallas_call(
        _l1_body,
        grid=grid,
        in_specs=[
            pl.BlockSpec((NC, _RB, D), lambda i: (0, i, 0)),
            pl.BlockSpec((_RB, D), lambda i: (i, 0)),
            pl.BlockSpec((D, 2 * D), lambda i: (0, 0)),
            pl.BlockSpec((D, 2 * D), lambda i: (0, 0)),
            pl.BlockSpec((1, 2 * D), lambda i: (0, 0)),
            pl.BlockSpec((1, 1), lambda i: (0, 0)),
        ],
        out_specs=pl.BlockSpec((NC, _RB, D), lambda i: (0, i, 0)),
        out_shape=jax.ShapeDtypeStruct((NC, N, D), jnp.float32),
    )(aggp, x, W_rel1, W_root1, b_rel1.reshape(1, -1), a1.reshape(1, 1))


def _tc_layer2(agg2, h1s, W_rel2, b_rel2, W_root2, a2, W_fc, b_fc):
    grid = (N // _RB,)
    return pl.pallas_call(
        _l2_body,
        grid=grid,
        in_specs=[
            pl.BlockSpec((NC, _RB, D), lambda i: (0, i, 0)),
            pl.BlockSpec((NC, _RB, D), lambda i: (0, i, 0)),
            pl.BlockSpec((2 * D, 2 * D), lambda i: (0, 0)),
            pl.BlockSpec((2 * D, 2 * D), lambda i: (0, 0)),
            pl.BlockSpec((1, 2 * D), lambda i: (0, 0)),
            pl.BlockSpec((1, 1), lambda i: (0, 0)),
            pl.BlockSpec((2 * D, D), lambda i: (0, 0)),
            pl.BlockSpec((1, D), lambda i: (0, 0)),
        ],
        out_specs=pl.BlockSpec((_RB, D), lambda i: (i, 0)),
        out_shape=jax.ShapeDtypeStruct((N, D), jnp.float32),
    )(agg2, h1s, W_rel2, W_root2, b_rel2.reshape(1, -1), a2.reshape(1, 1),
      W_fc, b_fc.reshape(1, -1))


def kernel(x, edge_index, edge_attr, W_rel1, b_rel1, W_root1, a1,
           W_rel2, b_rel2, W_root2, a2, W_fc, b_fc):
    E = edge_index.shape[1]
    src = edge_index[0].astype(jnp.int32)
    dst = edge_index[1].astype(jnp.int32)
    w = edge_attr.astype(jnp.float32)

    R = E // CHUNK
    G = 5
    ng1 = R // (NC * NS) // G
    ng2 = R // NS // G
    dst_m1 = dst.reshape(NC * NS, ng1, G, CHUNK)
    dst_m2 = dst.reshape(NS, ng2, G, CHUNK)
    wrep = jnp.broadcast_to(w[:, None], (E, 16))
    w_m1 = wrep.reshape(NC * NS, ng1, G * CHUNK // 8, D)
    w_m2 = wrep.reshape(NS, ng2, G * CHUNK // 8, D)
    src_same = jnp.stack([src, src]).reshape(NC, NC * NS, ng1, G, CHUNK)
    src_off = jnp.stack([src, src + N]).reshape(NC, NS, ng2, G, CHUNK)

    seg1 = _make_segsum(E, mode=1)
    seg2 = _make_segsum(E, mode=2)

    agg1p = seg1(x, src_same, dst_m1, w_m1)[:, :N]
    h1s = _tc_layer1(agg1p, x, W_rel1, b_rel1, W_root1, a1)
    agg2 = seg2(h1s.reshape(NC * N, D), src_off, dst_m2, w_m2)[:, :N]
    out = _tc_layer2(agg2, h1s, W_rel2, b_rel2, W_root2, a2, W_fc, b_fc)
    return out.reshape(BATCH, -1, D)


# speedup vs baseline: 4.6137x; 1.3423x over previous
import functools

import jax
import jax.numpy as jnp
from jax import lax
from jax.experimental import pallas as pl
from jax.experimental.pallas import tpu as pltpu
from jax.experimental.pallas import tpu_sc as plsc

N = 10000
NPAD = 10240
D = 128
CHUNK = 80
NC, NS = 2, 16
BATCH = 10


def _make_segsum(E, mode):
    R = E // CHUNK
    rows_pw = R // (NC * NS) if mode == 1 else R // NS
    G = 5
    ngroups = rows_pw // G
    rows_per_tile = NPAD // NS
    mesh = plsc.VectorSubcoreMesh(core_axis_name="c", subcore_axis_name="s")

    @functools.partial(
        pl.kernel,
        mesh=mesh,
        out_type=jax.ShapeDtypeStruct((NC, NPAD, D), jnp.float32),
        scratch_types=[
            pltpu.VMEM((G, CHUNK), jnp.int32),
            pltpu.VMEM((G, CHUNK), jnp.int32),
            pltpu.VMEM((G * CHUNK // 8, D), jnp.float32),
            pltpu.VMEM((CHUNK, D), jnp.float32),
            pltpu.VMEM((CHUNK, D), jnp.float32),
            pltpu.VMEM_SHARED((NPAD, D), jnp.float32),
            pltpu.SemaphoreType.DMA,
            pltpu.SemaphoreType.DMA,
            pltpu.SemaphoreType.DMA,
            pltpu.SemaphoreType.DMA,
        ],
    )
    def seg(table_h, srcs_h, dst_h, wrep_h, out_h,
            src_v, dst_v, wrep_v, rows_a, rows_b, agg_sh,
            gsem_a, gsem_b, ssem_a, ssem_b):
        c = lax.axis_index("c")
        s = lax.axis_index("s")
        wid = s * NC + c if mode == 1 else s
        rows = (rows_a, rows_b)
        gsem = (gsem_a, gsem_b)
        ssem = (ssem_a, ssem_b)

        zvec = jnp.zeros((16,), jnp.float32)

        def zrow(r, _):
            for k in range(D // 16):
                rows_a[r, pl.ds(k * 16, 16)] = zvec
            return 0

        lax.fori_loop(0, CHUNK, zrow, 0)

        def zcopy(i, _):
            pltpu.sync_copy(
                rows_a, agg_sh.at[pl.ds(s * rows_per_tile + i * CHUNK, CHUNK)])
            return 0

        lax.fori_loop(0, rows_per_tile // CHUNK, zcopy, 0)
        plsc.subcore_barrier()

        if mode == 1:
            off = None
        else:
            off = jnp.full((16,), c * N, jnp.int32)

        def group_body(g, _):
            pltpu.sync_copy(srcs_h.at[wid, g], src_v)
            pltpu.sync_copy(dst_h.at[wid, g], dst_v)
            pltpu.sync_copy(wrep_h.at[wid, g], wrep_v)
            if off is not None:
                def offs(j, _):
                    for t in range(CHUNK // 16):
                        src_v[j, pl.ds(t * 16, 16)] = (
                            src_v[j, pl.ds(t * 16, 16)] + off)
                    return 0
                lax.fori_loop(0, G, offs, 0)

            def scale(buf, j):
                def srow(r, _):
                    for rr in range(8):
                        e = r * 8 + rr
                        wsp = wrep_v[j * (CHUNK // 8) + r,
                                     pl.ds(rr * 16, 16)]
                        for k in range(D // 16):
                            buf[e, pl.ds(k * 16, 16)] = (
                                buf[e, pl.ds(k * 16, 16)] * wsp)
                    return 0
                lax.fori_loop(0, CHUNK // 8, srow, 0)

            gathers = [None, None]
            scatters = [None, None]
            gathers[0] = pltpu.async_copy(
                table_h.at[src_v.at[0]], rows[0], gsem[0])
            for j in range(G):
                b = j % 2
                nb = (j + 1) % 2
                if j + 1 < G:
                    if scatters[nb] is not None:
                        scatters[nb].wait()
                        scatters[nb] = None
                    gathers[nb] = pltpu.async_copy(
                        table_h.at[src_v.at[j + 1]], rows[nb], gsem[nb])
                gathers[b].wait()
                scale(rows[b], j)
                scatters[b] = pltpu.async_copy(
                    rows[b], agg_sh.at[dst_v.at[j]], ssem[b], add=True)
            for sc in scatters:
                if sc is not None:
                    sc.wait()
            return 0

        lax.fori_loop(0, ngroups, group_body, 0)
        plsc.subcore_barrier()

        pltpu.sync_copy(
            agg_sh.at[pl.ds(s * rows_per_tile, rows_per_tile)],
            out_h.at[c, pl.ds(s * rows_per_tile, rows_per_tile)])

    return seg


def _l1_body(aggp_ref, x_ref, wr_ref, wro_ref, b_ref, a_ref, out_ref):
    agg = aggp_ref[0] + aggp_ref[1]
    t = jnp.dot(agg, wr_ref[...], preferred_element_type=jnp.float32)
    t += jnp.dot(x_ref[...], wro_ref[...], preferred_element_type=jnp.float32)
    t += b_ref[...]
    h = jnp.where(t >= 0.0, t, a_ref[...] * t)
    out_ref[0] = h[:, :D]
    out_ref[1] = h[:, D:]


def _l2_body(agg_ref, h1_ref, wr_ref, wro_ref, b_ref, a_ref, wfc_ref,
             bfc_ref, out_ref):
    t = jnp.dot(agg_ref[0], wr_ref[:D], preferred_element_type=jnp.float32)
    t += jnp.dot(agg_ref[1], wr_ref[D:], preferred_element_type=jnp.float32)
    t += jnp.dot(h1_ref[0], wro_ref[:D], preferred_element_type=jnp.float32)
    t += jnp.dot(h1_ref[1], wro_ref[D:], preferred_element_type=jnp.float32)
    t += b_ref[...]
    h = jnp.where(t >= 0.0, t, a_ref[...] * t)
    out_ref[...] = (
        jnp.dot(h, wfc_ref[...], preferred_element_type=jnp.float32)
        + bfc_ref[...])


_RB = 1000


def _tc_layer1(aggp, x, W_rel1, b_rel1, W_root1, a1):
    grid = (N // _RB,)
    return pl.pallas_call(
        _l1_body,
        grid=grid,
        in_specs=[
            pl.BlockSpec((NC, _RB, D), lambda i: (0, i, 0)),
            pl.BlockSpec((_RB, D), lambda i: (i, 0)),
            pl.BlockSpec((D, 2 * D), lambda i: (0, 0)),
            pl.BlockSpec((D, 2 * D), lambda i: (0, 0)),
            pl.BlockSpec((1, 2 * D), lambda i: (0, 0)),
            pl.BlockSpec((1, 1), lambda i: (0, 0)),
        ],
        out_specs=pl.BlockSpec((NC, _RB, D), lambda i: (0, i, 0)),
        out_shape=jax.ShapeDtypeStruct((NC, N, D), jnp.float32),
    )(aggp, x, W_rel1, W_root1, b_rel1.reshape(1, -1), a1.reshape(1, 1))


def _tc_layer2(agg2, h1s, W_rel2, b_rel2, W_root2, a2, W_fc, b_fc):
    grid = (N // _RB,)
    return pl.pallas_call(
        _l2_body,
        grid=grid,
        in_specs=[
            pl.BlockSpec((NC, _RB, D), lambda i: (0, i, 0)),
            pl.BlockSpec((NC, _RB, D), lambda i: (0, i, 0)),
            pl.BlockSpec((2 * D, 2 * D), lambda i: (0, 0)),
            pl.BlockSpec((2 * D, 2 * D), lambda i: (0, 0)),
            pl.BlockSpec((1, 2 * D), lambda i: (0, 0)),
            pl.BlockSpec((1, 1), lambda i: (0, 0)),
            pl.BlockSpec((2 * D, D), lambda i: (0, 0)),
            pl.BlockSpec((1, D), lambda i: (0, 0)),
        ],
        out_specs=pl.BlockSpec((_RB, D), lambda i: (i, 0)),
        out_shape=jax.ShapeDtypeStruct((N, D), jnp.float32),
    )(agg2, h1s, W_rel2, W_root2, b_rel2.reshape(1, -1), a2.reshape(1, 1),
      W_fc, b_fc.reshape(1, -1))


def kernel(x, edge_index, edge_attr, W_rel1, b_rel1, W_root1, a1,
           W_rel2, b_rel2, W_root2, a2, W_fc, b_fc):
    E = edge_index.shape[1]
    src = edge_index[0].astype(jnp.int32)
    dst = edge_index[1].astype(jnp.int32)
    w = edge_attr.astype(jnp.float32)

    R = E // CHUNK
    G = 5
    ng1 = R // (NC * NS) // G
    ng2 = R // NS // G
    dst_m1 = dst.reshape(NC * NS, ng1, G, CHUNK)
    dst_m2 = dst.reshape(NS, ng2, G, CHUNK)
    wrep = jnp.broadcast_to(w[:, None], (E, 16))
    w_m1 = wrep.reshape(NC * NS, ng1, G * CHUNK // 8, D)
    w_m2 = wrep.reshape(NS, ng2, G * CHUNK // 8, D)
    src_m1 = src.reshape(NC * NS, ng1, G, CHUNK)
    src_m2 = src.reshape(NS, ng2, G, CHUNK)

    seg1 = _make_segsum(E, mode=1)
    seg2 = _make_segsum(E, mode=2)

    agg1p = seg1(x, src_m1, dst_m1, w_m1)[:, :N]
    h1s = _tc_layer1(agg1p, x, W_rel1, b_rel1, W_root1, a1)
    agg2 = seg2(h1s.reshape(NC * N, D), src_m2, dst_m2, w_m2)[:, :N]
    out = _tc_layer2(agg2, h1s, W_rel2, b_rel2, W_root2, a2, W_fc, b_fc)
    return out.reshape(BATCH, -1, D)


# speedup vs baseline: 5.2370x; 1.1351x over previous
import functools

import jax
import jax.numpy as jnp
from jax import lax
from jax.experimental import pallas as pl
from jax.experimental.pallas import tpu as pltpu
from jax.experimental.pallas import tpu_sc as plsc

N = 10000
NPAD = 10240
D = 128
CHUNK = 80
NC, NS = 2, 16
BATCH = 10


def _make_segsum(E, mode):
    R = E // CHUNK
    rows_pw = R // (NC * NS) if mode == 1 else R // NS
    G = 5
    ngroups = rows_pw // G
    rows_per_tile = NPAD // NS
    mesh = plsc.VectorSubcoreMesh(core_axis_name="c", subcore_axis_name="s")

    @functools.partial(
        pl.kernel,
        mesh=mesh,
        out_type=jax.ShapeDtypeStruct((NC, NPAD, D), jnp.float32),
        scratch_types=[
            pltpu.VMEM((G, CHUNK), jnp.int32),
            pltpu.VMEM((G, CHUNK), jnp.int32),
            pltpu.VMEM((G * CHUNK,), jnp.float32),
            pltpu.VMEM((CHUNK, D), jnp.float32),
            pltpu.VMEM((CHUNK, D), jnp.float32),
            pltpu.VMEM_SHARED((NPAD, D), jnp.float32),
            pltpu.SemaphoreType.DMA,
            pltpu.SemaphoreType.DMA,
            pltpu.SemaphoreType.DMA,
            pltpu.SemaphoreType.DMA,
        ],
    )
    def seg(table_h, srcs_h, dst_h, w_h, out_h,
            src_v, dst_v, w_v, rows_a, rows_b, agg_sh,
            gsem_a, gsem_b, ssem_a, ssem_b):
        c = lax.axis_index("c")
        s = lax.axis_index("s")
        wid = s * NC + c if mode == 1 else s
        rows = (rows_a, rows_b)
        gsem = (gsem_a, gsem_b)
        ssem = (ssem_a, ssem_b)

        zvec = jnp.zeros((16,), jnp.float32)

        def zrow(r, _):
            for k in range(D // 16):
                rows_a[r, pl.ds(k * 16, 16)] = zvec
            return 0

        lax.fori_loop(0, CHUNK, zrow, 0)

        def zcopy(i, _):
            pltpu.sync_copy(
                rows_a, agg_sh.at[pl.ds(s * rows_per_tile + i * CHUNK, CHUNK)])
            return 0

        lax.fori_loop(0, rows_per_tile // CHUNK, zcopy, 0)
        plsc.subcore_barrier()

        if mode == 1:
            off = None
        else:
            off = jnp.full((16,), c * N, jnp.int32)

        def group_body(g, _):
            pltpu.sync_copy(srcs_h.at[wid, g], src_v)
            pltpu.sync_copy(dst_h.at[wid, g], dst_v)
            pltpu.sync_copy(w_h.at[wid, g], w_v)
            if off is not None:
                def offs(j, _):
                    for t in range(CHUNK // 16):
                        src_v[j, pl.ds(t * 16, 16)] = (
                            src_v[j, pl.ds(t * 16, 16)] + off)
                    return 0
                lax.fori_loop(0, G, offs, 0)

            def scale(buf, j):
                @plsc.parallel_loop(0, CHUNK // 16)
                def srow(r):
                    wv = w_v[pl.ds(j * CHUNK + r * 16, 16)]
                    for rr in range(16):
                        e = r * 16 + rr
                        wsp = jnp.full((16,), wv[rr], jnp.float32)
                        for k in range(D // 16):
                            buf[e, pl.ds(k * 16, 16)] = (
                                buf[e, pl.ds(k * 16, 16)] * wsp)

            gathers = [None, None]
            scatters = [None, None]
            gathers[0] = pltpu.async_copy(
                table_h.at[src_v.at[0]], rows[0], gsem[0])
            for j in range(G):
                b = j % 2
                nb = (j + 1) % 2
                if j + 1 < G:
                    if scatters[nb] is not None:
                        scatters[nb].wait()
                        scatters[nb] = None
                    gathers[nb] = pltpu.async_copy(
                        table_h.at[src_v.at[j + 1]], rows[nb], gsem[nb])
                gathers[b].wait()
                scale(rows[b], j)
                scatters[b] = pltpu.async_copy(
                    rows[b], agg_sh.at[dst_v.at[j]], ssem[b], add=True)
            for sc in scatters:
                if sc is not None:
                    sc.wait()
            return 0

        lax.fori_loop(0, ngroups, group_body, 0)
        plsc.subcore_barrier()

        pltpu.sync_copy(
            agg_sh.at[pl.ds(s * rows_per_tile, rows_per_tile)],
            out_h.at[c, pl.ds(s * rows_per_tile, rows_per_tile)])

    return seg


def _l1_body(aggp_ref, x_ref, wr_ref, wro_ref, b_ref, a_ref, out_ref):
    agg = aggp_ref[0] + aggp_ref[1]
    t = jnp.dot(agg, wr_ref[...], preferred_element_type=jnp.float32)
    t += jnp.dot(x_ref[...], wro_ref[...], preferred_element_type=jnp.float32)
    t += b_ref[...]
    h = jnp.where(t >= 0.0, t, a_ref[...] * t)
    out_ref[0] = h[:, :D]
    out_ref[1] = h[:, D:]


def _l2_body(agg_ref, h1_ref, wr_ref, wro_ref, b_ref, a_ref, wfc_ref,
             bfc_ref, out_ref):
    t = jnp.dot(agg_ref[0], wr_ref[:D], preferred_element_type=jnp.float32)
    t += jnp.dot(agg_ref[1], wr_ref[D:], preferred_element_type=jnp.float32)
    t += jnp.dot(h1_ref[0], wro_ref[:D], preferred_element_type=jnp.float32)
    t += jnp.dot(h1_ref[1], wro_ref[D:], preferred_element_type=jnp.float32)
    t += b_ref[...]
    h = jnp.where(t >= 0.0, t, a_ref[...] * t)
    out_ref[...] = (
        jnp.dot(h, wfc_ref[...], preferred_element_type=jnp.float32)
        + bfc_ref[...])


_RB = 1000


def _tc_layer1(aggp, x, W_rel1, b_rel1, W_root1, a1):
    grid = (N // _RB,)
    return pl.pallas_call(
        _l1_body,
        grid=grid,
        in_specs=[
            pl.BlockSpec((NC, _RB, D), lambda i: (0, i, 0)),
            pl.BlockSpec((_RB, D), lambda i: (i, 0)),
            pl.BlockSpec((D, 2 * D), lambda i: (0, 0)),
            pl.BlockSpec((D, 2 * D), lambda i: (0, 0)),
            pl.BlockSpec((1, 2 * D), lambda i: (0, 0)),
            pl.BlockSpec((1, 1), lambda i: (0, 0)),
        ],
        out_specs=pl.BlockSpec((NC, _RB, D), lambda i: (0, i, 0)),
        out_shape=jax.ShapeDtypeStruct((NC, N, D), jnp.float32),
    )(aggp, x, W_rel1, W_root1, b_rel1.reshape(1, -1), a1.reshape(1, 1))


def _tc_layer2(agg2, h1s, W_rel2, b_rel2, W_root2, a2, W_fc, b_fc):
    grid = (N // _RB,)
    return pl.pallas_call(
        _l2_body,
        grid=grid,
        in_specs=[
            pl.BlockSpec((NC, _RB, D), lambda i: (0, i, 0)),
            pl.BlockSpec((NC, _RB, D), lambda i: (0, i, 0)),
            pl.BlockSpec((2 * D, 2 * D), lambda i: (0, 0)),
            pl.BlockSpec((2 * D, 2 * D), lambda i: (0, 0)),
            pl.BlockSpec((1, 2 * D), lambda i: (0, 0)),
            pl.BlockSpec((1, 1), lambda i: (0, 0)),
            pl.BlockSpec((2 * D, D), lambda i: (0, 0)),
            pl.BlockSpec((1, D), lambda i: (0, 0)),
        ],
        out_specs=pl.BlockSpec((_RB, D), lambda i: (i, 0)),
        out_shape=jax.ShapeDtypeStruct((N, D), jnp.float32),
    )(agg2, h1s, W_rel2, W_root2, b_rel2.reshape(1, -1), a2.reshape(1, 1),
      W_fc, b_fc.reshape(1, -1))


def kernel(x, edge_index, edge_attr, W_rel1, b_rel1, W_root1, a1,
           W_rel2, b_rel2, W_root2, a2, W_fc, b_fc):
    E = edge_index.shape[1]
    src = edge_index[0].astype(jnp.int32)
    dst = edge_index[1].astype(jnp.int32)
    w = edge_attr.astype(jnp.float32)

    R = E // CHUNK
    G = 5
    ng1 = R // (NC * NS) // G
    ng2 = R // NS // G
    dst_m1 = dst.reshape(NC * NS, ng1, G, CHUNK)
    dst_m2 = dst.reshape(NS, ng2, G, CHUNK)
    w_m1 = w.reshape(NC * NS, ng1, G * CHUNK)
    w_m2 = w.reshape(NS, ng2, G * CHUNK)
    src_m1 = src.reshape(NC * NS, ng1, G, CHUNK)
    src_m2 = src.reshape(NS, ng2, G, CHUNK)

    seg1 = _make_segsum(E, mode=1)
    seg2 = _make_segsum(E, mode=2)

    agg1p = seg1(x, src_m1, dst_m1, w_m1)[:, :N]
    h1s = _tc_layer1(agg1p, x, W_rel1, b_rel1, W_root1, a1)
    agg2 = seg2(h1s.reshape(NC * N, D), src_m2, dst_m2, w_m2)[:, :N]
    out = _tc_layer2(agg2, h1s, W_rel2, b_rel2, W_root2, a2, W_fc, b_fc)
    return out.reshape(BATCH, -1, D)


# speedup vs baseline: 7.6395x; 1.4588x over previous
import functools

import jax
import jax.numpy as jnp
from jax import lax
from jax.experimental import pallas as pl
from jax.experimental.pallas import tpu as pltpu
from jax.experimental.pallas import tpu_sc as plsc

N = 10000
NPAD = 10240
D = 128
CHUNK = 80
NC, NS = 2, 16
BATCH = 10


def _make_segsum(E, mode):
    R = E // CHUNK
    rows_pw = R // (NC * NS) if mode == 1 else R // NS
    G = 25
    GE = G * CHUNK
    ngroups = rows_pw // G
    NBODY = (G - 1) // 3
    rows_per_tile = NPAD // NS
    mesh = plsc.VectorSubcoreMesh(core_axis_name="c", subcore_axis_name="s")

    @functools.partial(
        pl.kernel,
        mesh=mesh,
        out_type=jax.ShapeDtypeStruct((NC, NPAD, D), jnp.float32),
        scratch_types=[
            pltpu.VMEM((GE,), jnp.int32),
            pltpu.VMEM((G, CHUNK), jnp.int32),
            pltpu.VMEM((GE,), jnp.float32),
            pltpu.VMEM((CHUNK, D), jnp.float32),
            pltpu.VMEM((CHUNK, D), jnp.float32),
            pltpu.VMEM((CHUNK, D), jnp.float32),
            pltpu.VMEM_SHARED((NPAD, D), jnp.float32),
            pltpu.SemaphoreType.DMA,
            pltpu.SemaphoreType.DMA,
            pltpu.SemaphoreType.DMA,
            pltpu.SemaphoreType.DMA,
            pltpu.SemaphoreType.DMA,
            pltpu.SemaphoreType.DMA,
            pltpu.SemaphoreType.DMA,
        ],
    )
    def seg(table_h, src_h, dst_h, w_h, out_h,
            src_v, dst_v, w_v, rows_0, rows_1, rows_2, agg_sh,
            gsem_0, gsem_1, gsem_2, ssem_0, ssem_1, ssem_2, tsem):
        c = lax.axis_index("c")
        s = lax.axis_index("s")
        wid = s * NC + c if mode == 1 else s
        wbase = wid * ngroups
        ebase = wid * rows_pw * CHUNK
        rows = (rows_0, rows_1, rows_2)
        gsem = (gsem_0, gsem_1, gsem_2)
        ssem = (ssem_0, ssem_1, ssem_2)
        rows_a = rows_0

        zvec = jnp.zeros((16,), jnp.float32)

        def zrow(r, _):
            for k in range(D // 16):
                rows_a[r, pl.ds(k * 16, 16)] = zvec
            return 0

        lax.fori_loop(0, CHUNK, zrow, 0)

        def zcopy(i, _):
            pltpu.sync_copy(
                rows_a, agg_sh.at[pl.ds(s * rows_per_tile + i * CHUNK, CHUNK)])
            return 0

        lax.fori_loop(0, rows_per_tile // CHUNK, zcopy, 0)
        plsc.subcore_barrier()

        if mode == 1:
            off = None
        else:
            off = jnp.full((16,), c * N, jnp.int32)

        def scale(buf, j):
            @plsc.parallel_loop(0, CHUNK // 16)
            def srow(r):
                wv = w_v[pl.ds(j * CHUNK + r * 16, 16)]
                for rr in range(16):
                    e = r * 16 + rr
                    wsp = jnp.full((16,), wv[rr], jnp.float32)
                    for k in range(D // 16):
                        buf[e, pl.ds(k * 16, 16)] = (
                            buf[e, pl.ds(k * 16, 16)] * wsp)

        def gather(j, b):
            return pltpu.async_copy(
                table_h.at[src_v.at[pl.ds(j * CHUNK, CHUNK)]],
                rows[b], gsem[b])

        def gather_wait(b):
            pltpu.make_async_copy(
                table_h.at[src_v.at[pl.ds(0, CHUNK)]],
                rows[b], gsem[b]).wait()

        def scatter(j, b):
            return pltpu.async_copy(
                rows[b], agg_sh.at[dst_v.at[j]], ssem[b], add=True)

        def scatter_wait(b):
            pltpu.make_async_copy(
                rows[b], agg_sh.at[dst_v.at[0]], ssem[b]).wait()

        def group_body(g, _):
            pltpu.async_copy(src_h.at[pl.ds(ebase + g * GE, GE)], src_v, tsem)
            pltpu.async_copy(dst_h.at[wbase + g], dst_v, tsem)
            pltpu.async_copy(w_h.at[pl.ds(ebase + g * GE, GE)], w_v, tsem)
            pltpu.make_async_copy(
                src_h.at[pl.ds(ebase + g * GE, GE)], src_v, tsem).wait()
            pltpu.make_async_copy(dst_h.at[wbase + g], dst_v, tsem).wait()
            pltpu.make_async_copy(
                w_h.at[pl.ds(ebase + g * GE, GE)], w_v, tsem).wait()
            if off is not None:
                @plsc.parallel_loop(0, GE // 16)
                def offs(t):
                    src_v[pl.ds(t * 16, 16)] = (
                        src_v[pl.ds(t * 16, 16)] + off)

            g0 = gather(0, 0)
            gather(1, 1)
            g0.wait()
            scale(rows[0], 0)
            scatter(0, 0)
            gather(2, 2)

            def triple(t, _):
                c1 = 3 * t + 1
                gather_wait(1)
                scale(rows[1], c1)
                s1 = scatter(c1, 1)
                scatter_wait(0)
                gather(c1 + 2, 0)

                gather_wait(2)
                scale(rows[2], c1 + 1)
                s2 = scatter(c1 + 1, 2)
                s1.wait()

                @pl.when(t < NBODY - 1)
                def _():
                    gather(c1 + 3, 1)

                gather_wait(0)
                scale(rows[0], c1 + 2)
                scatter(c1 + 2, 0)
                s2.wait()

                @pl.when(t < NBODY - 1)
                def _():
                    gather(c1 + 4, 2)
                return 0

            lax.fori_loop(0, NBODY, triple, 0)
            scatter_wait(0)
            return 0

        lax.fori_loop(0, ngroups, group_body, 0)
        plsc.subcore_barrier()

        pltpu.sync_copy(
            agg_sh.at[pl.ds(s * rows_per_tile, rows_per_tile)],
            out_h.at[c, pl.ds(s * rows_per_tile, rows_per_tile)])

    return seg


def _l1_body(aggp_ref, x_ref, wr_ref, wro_ref, b_ref, a_ref, out_ref):
    agg = aggp_ref[0] + aggp_ref[1]
    t = jnp.dot(agg, wr_ref[...], preferred_element_type=jnp.float32)
    t += jnp.dot(x_ref[...], wro_ref[...], preferred_element_type=jnp.float32)
    t += b_ref[...]
    h = jnp.where(t >= 0.0, t, a_ref[...] * t)
    out_ref[0] = h[:, :D]
    out_ref[1] = h[:, D:]


def _l2_body(agg_ref, h1_ref, wr_ref, wro_ref, b_ref, a_ref, wfc_ref,
             bfc_ref, out_ref):
    t = jnp.dot(agg_ref[0], wr_ref[:D], preferred_element_type=jnp.float32)
    t += jnp.dot(agg_ref[1], wr_ref[D:], preferred_element_type=jnp.float32)
    t += jnp.dot(h1_ref[0], wro_ref[:D], preferred_element_type=jnp.float32)
    t += jnp.dot(h1_ref[1], wro_ref[D:], preferred_element_type=jnp.float32)
    t += b_ref[...]
    h = jnp.where(t >= 0.0, t, a_ref[...] * t)
    out_ref[...] = (
        jnp.dot(h, wfc_ref[...], preferred_element_type=jnp.float32)
        + bfc_ref[...])


_RB = 1000


def _tc_layer1(aggp, x, W_rel1, b_rel1, W_root1, a1):
    grid = (N // _RB,)
    return pl.pallas_call(
        _l1_body,
        grid=grid,
        in_specs=[
            pl.BlockSpec((NC, _RB, D), lambda i: (0, i, 0)),
            pl.BlockSpec((_RB, D), lambda i: (i, 0)),
            pl.BlockSpec((D, 2 * D), lambda i: (0, 0)),
            pl.BlockSpec((D, 2 * D), lambda i: (0, 0)),
            pl.BlockSpec((1, 2 * D), lambda i: (0, 0)),
            pl.BlockSpec((1, 1), lambda i: (0, 0)),
        ],
        out_specs=pl.BlockSpec((NC, _RB, D), lambda i: (0, i, 0)),
        out_shape=jax.ShapeDtypeStruct((NC, N, D), jnp.float32),
    )(aggp, x, W_rel1, W_root1, b_rel1.reshape(1, -1), a1.reshape(1, 1))


def _tc_layer2(agg2, h1s, W_rel2, b_rel2, W_root2, a2, W_fc, b_fc):
    grid = (N // _RB,)
    return pl.pallas_call(
        _l2_body,
        grid=grid,
        in_specs=[
            pl.BlockSpec((NC, _RB, D), lambda i: (0, i, 0)),
            pl.BlockSpec((NC, _RB, D), lambda i: (0, i, 0)),
            pl.BlockSpec((2 * D, 2 * D), lambda i: (0, 0)),
            pl.BlockSpec((2 * D, 2 * D), lambda i: (0, 0)),
            pl.BlockSpec((1, 2 * D), lambda i: (0, 0)),
            pl.BlockSpec((1, 1), lambda i: (0, 0)),
            pl.BlockSpec((2 * D, D), lambda i: (0, 0)),
            pl.BlockSpec((1, D), lambda i: (0, 0)),
        ],
        out_specs=pl.BlockSpec((_RB, D), lambda i: (i, 0)),
        out_shape=jax.ShapeDtypeStruct((N, D), jnp.float32),
    )(agg2, h1s, W_rel2, W_root2, b_rel2.reshape(1, -1), a2.reshape(1, 1),
      W_fc, b_fc.reshape(1, -1))


def kernel(x, edge_index, edge_attr, W_rel1, b_rel1, W_root1, a1,
           W_rel2, b_rel2, W_root2, a2, W_fc, b_fc):
    E = edge_index.shape[1]
    src = edge_index[0].astype(jnp.int32)
    dst = edge_index[1].astype(jnp.int32)
    w = edge_attr.astype(jnp.float32)

    G = 25
    dst3 = dst.reshape(E // (G * CHUNK), G, CHUNK)

    seg1 = _make_segsum(E, mode=1)
    seg2 = _make_segsum(E, mode=2)

    agg1p = seg1(x, src, dst3, w)[:, :N]
    h1s = _tc_layer1(agg1p, x, W_rel1, b_rel1, W_root1, a1)
    agg2 = seg2(h1s.reshape(NC * N, D), src, dst3, w)[:, :N]
    out = _tc_layer2(agg2, h1s, W_rel2, b_rel2, W_root2, a2, W_fc, b_fc)
    return out.reshape(BATCH, -1, D)


# speedup vs baseline: 7.8393x; 1.0261x over previous
import functools

import jax
import jax.numpy as jnp
from jax import lax
from jax.experimental import pallas as pl
from jax.experimental.pallas import tpu as pltpu
from jax.experimental.pallas import tpu_sc as plsc

N = 10000
NPAD = 10240
D = 128
CHUNK = 80
NC, NS = 2, 16
BATCH = 10


def _make_segsum(E, mode):
    R = E // CHUNK
    rows_pw = R // (NC * NS) if mode == 1 else R // NS
    G = 25
    GE = G * CHUNK
    ngroups = rows_pw // G
    NBODY = (G - 1) // 3
    rows_per_tile = NPAD // NS
    mesh = plsc.VectorSubcoreMesh(core_axis_name="c", subcore_axis_name="s")

    @functools.partial(
        pl.kernel,
        mesh=mesh,
        out_type=jax.ShapeDtypeStruct((NC, NPAD, D), jnp.float32),
        scratch_types=[
            pltpu.VMEM((GE,), jnp.int32),
            pltpu.VMEM((G, CHUNK), jnp.int32),
            pltpu.VMEM((GE,), jnp.float32),
            pltpu.VMEM((CHUNK, D), jnp.float32),
            pltpu.VMEM((CHUNK, D), jnp.float32),
            pltpu.VMEM((CHUNK, D), jnp.float32),
            pltpu.VMEM_SHARED((NPAD, D), jnp.float32),
            pltpu.SemaphoreType.DMA,
            pltpu.SemaphoreType.DMA,
            pltpu.SemaphoreType.DMA,
            pltpu.SemaphoreType.DMA,
            pltpu.SemaphoreType.DMA,
            pltpu.SemaphoreType.DMA,
            pltpu.SemaphoreType.DMA,
        ],
    )
    def seg(table_h, src_h, dst_h, w_h, out_h,
            src_v, dst_v, w_v, rows_0, rows_1, rows_2, agg_sh,
            gsem_0, gsem_1, gsem_2, ssem_0, ssem_1, ssem_2, tsem):
        c = lax.axis_index("c")
        s = lax.axis_index("s")
        wid = s * NC + c if mode == 1 else s
        wbase = wid * ngroups
        ebase = wid * rows_pw * CHUNK
        rows = (rows_0, rows_1, rows_2)
        gsem = (gsem_0, gsem_1, gsem_2)
        ssem = (ssem_0, ssem_1, ssem_2)
        rows_a = rows_0

        zvec = jnp.zeros((16,), jnp.float32)

        def zrow(r, _):
            for k in range(D // 16):
                rows_a[r, pl.ds(k * 16, 16)] = zvec
            return 0

        lax.fori_loop(0, CHUNK, zrow, 0)

        def zcopy(i, _):
            pltpu.sync_copy(
                rows_a, agg_sh.at[pl.ds(s * rows_per_tile + i * CHUNK, CHUNK)])
            return 0

        lax.fori_loop(0, rows_per_tile // CHUNK, zcopy, 0)
        plsc.subcore_barrier()

        if mode == 1:
            off = None
        else:
            off = jnp.full((16,), c * N, jnp.int32)

        def scale(buf, j):
            @plsc.parallel_loop(0, CHUNK // 16)
            def srow(r):
                wv = w_v[pl.ds(j * CHUNK + r * 16, 16)]
                for rr in range(16):
                    e = r * 16 + rr
                    wsp = jnp.full((16,), wv[rr], jnp.float32)
                    for k in range(D // 16):
                        buf[e, pl.ds(k * 16, 16)] = (
                            buf[e, pl.ds(k * 16, 16)] * wsp)

        def gather(j, b):
            return pltpu.async_copy(
                table_h.at[src_v.at[pl.ds(j * CHUNK, CHUNK)]],
                rows[b], gsem[b])

        def gather_wait(b):
            pltpu.make_async_copy(
                table_h.at[src_v.at[pl.ds(0, CHUNK)]],
                rows[b], gsem[b]).wait()

        def scatter(j, b):
            return pltpu.async_copy(
                rows[b], agg_sh.at[dst_v.at[j]], ssem[b], add=True)

        def scatter_wait(b):
            pltpu.make_async_copy(
                rows[b], agg_sh.at[dst_v.at[0]], ssem[b]).wait()

        def group_body(g, _):
            pltpu.async_copy(src_h.at[pl.ds(ebase + g * GE, GE)], src_v, tsem)
            pltpu.async_copy(dst_h.at[wbase + g], dst_v, tsem)
            pltpu.async_copy(w_h.at[pl.ds(ebase + g * GE, GE)], w_v, tsem)
            pltpu.make_async_copy(
                src_h.at[pl.ds(ebase + g * GE, GE)], src_v, tsem).wait()
            pltpu.make_async_copy(dst_h.at[wbase + g], dst_v, tsem).wait()
            pltpu.make_async_copy(
                w_h.at[pl.ds(ebase + g * GE, GE)], w_v, tsem).wait()
            if off is not None:
                @plsc.parallel_loop(0, GE // 16)
                def offs(t):
                    src_v[pl.ds(t * 16, 16)] = (
                        src_v[pl.ds(t * 16, 16)] + off)

            g0 = gather(0, 0)
            gather(1, 1)
            g0.wait()
            scale(rows[0], 0)
            scatter(0, 0)
            gather(2, 2)

            def triple(t, _):
                c1 = 3 * t + 1
                gather_wait(1)
                scale(rows[1], c1)
                s1 = scatter(c1, 1)
                scatter_wait(0)
                gather(c1 + 2, 0)

                gather_wait(2)
                scale(rows[2], c1 + 1)
                s2 = scatter(c1 + 1, 2)
                s1.wait()

                @pl.when(t < NBODY - 1)
                def _():
                    gather(c1 + 3, 1)

                gather_wait(0)
                scale(rows[0], c1 + 2)
                scatter(c1 + 2, 0)
                s2.wait()

                @pl.when(t < NBODY - 1)
                def _():
                    gather(c1 + 4, 2)
                return 0

            lax.fori_loop(0, NBODY, triple, 0)
            scatter_wait(0)
            return 0

        lax.fori_loop(0, ngroups, group_body, 0)
        plsc.subcore_barrier()

        pltpu.sync_copy(
            agg_sh.at[pl.ds(s * rows_per_tile, rows_per_tile)],
            out_h.at[c, pl.ds(s * rows_per_tile, rows_per_tile)])

    return seg


def _l1_body(aggp_ref, x_ref, wr_ref, wro_ref, b_ref, a_ref, out_ref):
    agg = aggp_ref[0] + aggp_ref[1]
    t = jnp.dot(agg, wr_ref[...], preferred_element_type=jnp.float32)
    t += jnp.dot(x_ref[...], wro_ref[...], preferred_element_type=jnp.float32)
    t += b_ref[...]
    h = jnp.where(t >= 0.0, t, a_ref[...] * t)
    out_ref[0] = h[:, :D]
    out_ref[1] = h[:, D:]


def _l2_body(agg_ref, h1_ref, wr_ref, wro_ref, b_ref, a_ref, wfc_ref,
             bfc_ref, out_ref):
    t = jnp.dot(agg_ref[0], wr_ref[:D], preferred_element_type=jnp.float32)
    t += jnp.dot(agg_ref[1], wr_ref[D:], preferred_element_type=jnp.float32)
    t += jnp.dot(h1_ref[0], wro_ref[:D], preferred_element_type=jnp.float32)
    t += jnp.dot(h1_ref[1], wro_ref[D:], preferred_element_type=jnp.float32)
    t += b_ref[...]
    h = jnp.where(t >= 0.0, t, a_ref[...] * t)
    out_ref[...] = (
        jnp.dot(h, wfc_ref[...], preferred_element_type=jnp.float32)
        + bfc_ref[...])


_RB = 1000


def _tc_layer1(aggp, x, W_rel1, b_rel1, W_root1, a1):
    grid = (N // _RB,)
    return pl.pallas_call(
        _l1_body,
        grid=grid,
        in_specs=[
            pl.BlockSpec((NC, _RB, D), lambda i: (0, i, 0)),
            pl.BlockSpec((_RB, D), lambda i: (i, 0)),
            pl.BlockSpec((D, 2 * D), lambda i: (0, 0)),
            pl.BlockSpec((D, 2 * D), lambda i: (0, 0)),
            pl.BlockSpec((1, 2 * D), lambda i: (0, 0)),
            pl.BlockSpec((1, 1), lambda i: (0, 0)),
        ],
        out_specs=pl.BlockSpec((NC, _RB, D), lambda i: (0, i, 0)),
        out_shape=jax.ShapeDtypeStruct((NC, N, D), jnp.float32),
    )(aggp, x, W_rel1, W_root1, b_rel1.reshape(1, -1), a1.reshape(1, 1))


def _tc_layer2(agg2, h1s, W_rel2, b_rel2, W_root2, a2, W_fc, b_fc):
    grid = (N // _RB,)
    return pl.pallas_call(
        _l2_body,
        grid=grid,
        in_specs=[
            pl.BlockSpec((NC, _RB, D), lambda i: (0, i, 0)),
            pl.BlockSpec((NC, _RB, D), lambda i: (0, i, 0)),
            pl.BlockSpec((2 * D, 2 * D), lambda i: (0, 0)),
            pl.BlockSpec((2 * D, 2 * D), lambda i: (0, 0)),
            pl.BlockSpec((1, 2 * D), lambda i: (0, 0)),
            pl.BlockSpec((1, 1), lambda i: (0, 0)),
            pl.BlockSpec((2 * D, D), lambda i: (0, 0)),
            pl.BlockSpec((1, D), lambda i: (0, 0)),
        ],
        out_specs=pl.BlockSpec((_RB, D), lambda i: (i, 0)),
        out_shape=jax.ShapeDtypeStruct((N, D), jnp.float32),
    )(agg2, h1s, W_rel2, W_root2, b_rel2.reshape(1, -1), a2.reshape(1, 1),
      W_fc, b_fc.reshape(1, -1))


def kernel(x, edge_index, edge_attr, W_rel1, b_rel1, W_root1, a1,
           W_rel2, b_rel2, W_root2, a2, W_fc, b_fc):
    E = edge_index.shape[1]
    src = edge_index[0].astype(jnp.int32)
    dst = edge_index[1].astype(jnp.int32)
    w = edge_attr.astype(jnp.float32)

    G = 25
    dst3 = dst.reshape(E // (G * CHUNK), G, CHUNK)

    seg1 = _make_segsum(E, mode=1)
    seg2 = _make_segsum(E, mode=2)

    agg1p = seg1(x, src, dst3, w)
    h1s = _tc_layer1(agg1p, x, W_rel1, b_rel1, W_root1, a1)
    agg2 = seg2(h1s.reshape(NC * N, D), src, dst3, w)
    out = _tc_layer2(agg2, h1s, W_rel2, b_rel2, W_root2, a2, W_fc, b_fc)
    return out.reshape(BATCH, -1, D)


# speedup vs baseline: 7.9624x; 1.0157x over previous
import functools

import jax
import jax.numpy as jnp
from jax import lax
from jax.experimental import pallas as pl
from jax.experimental.pallas import tpu as pltpu
from jax.experimental.pallas import tpu_sc as plsc

N = 10000
NPAD = 10240
D = 128
CHUNK = 80
NC, NS = 2, 16
BATCH = 10


def _make_segsum(E, mode):
    R = E // CHUNK
    rows_pw = R // (NC * NS) if mode == 1 else R // NS
    G = 25
    GE = G * CHUNK
    ngroups = rows_pw // G
    NBODY = (G - 1) // 3
    rows_per_tile = NPAD // NS
    mesh = plsc.VectorSubcoreMesh(core_axis_name="c", subcore_axis_name="s")

    @functools.partial(
        pl.kernel,
        mesh=mesh,
        out_type=jax.ShapeDtypeStruct((NC, NPAD, D), jnp.float32),
        scratch_types=[
            pltpu.VMEM((GE,), jnp.int32),
            pltpu.VMEM((GE,), jnp.int32),
            pltpu.VMEM((G, CHUNK), jnp.int32),
            pltpu.VMEM((GE,), jnp.float32),
            pltpu.VMEM((CHUNK, D), jnp.float32),
            pltpu.VMEM((CHUNK, D), jnp.float32),
            pltpu.VMEM((CHUNK, D), jnp.float32),
            pltpu.VMEM_SHARED((NPAD, D), jnp.float32),
            pltpu.SemaphoreType.DMA,
            pltpu.SemaphoreType.DMA,
            pltpu.SemaphoreType.DMA,
            pltpu.SemaphoreType.DMA,
            pltpu.SemaphoreType.DMA,
            pltpu.SemaphoreType.DMA,
            pltpu.SemaphoreType.DMA,
        ],
    )
    def seg(table_h, src_h, dst_h, w_h, out_h,
            src_v, dst1_v, dst_v, w_v, rows_0, rows_1, rows_2, agg_sh,
            gsem_0, gsem_1, gsem_2, ssem_0, ssem_1, ssem_2, tsem):
        c = lax.axis_index("c")
        s = lax.axis_index("s")
        wid = s * NC + c if mode == 1 else s
        wbase = wid * ngroups
        ebase = wid * rows_pw * CHUNK
        rows = (rows_0, rows_1, rows_2)
        gsem = (gsem_0, gsem_1, gsem_2)
        ssem = (ssem_0, ssem_1, ssem_2)
        rows_a = rows_0

        zvec = jnp.zeros((16,), jnp.float32)

        def zrow(r, _):
            for k in range(D // 16):
                rows_a[r, pl.ds(k * 16, 16)] = zvec
            return 0

        lax.fori_loop(0, CHUNK, zrow, 0)

        def zcopy(i, _):
            pltpu.sync_copy(
                rows_a, agg_sh.at[pl.ds(s * rows_per_tile + i * CHUNK, CHUNK)])
            return 0

        lax.fori_loop(0, rows_per_tile // CHUNK, zcopy, 0)
        plsc.subcore_barrier()

        if mode == 1:
            off = None
        else:
            off = jnp.full((16,), c * N, jnp.int32)

        def scale(buf, j):
            @plsc.parallel_loop(0, CHUNK // 16)
            def srow(r):
                wv = w_v[pl.ds(j * CHUNK + r * 16, 16)]
                for rr in range(16):
                    e = r * 16 + rr
                    wsp = jnp.full((16,), wv[rr], jnp.float32)
                    for k in range(D // 16):
                        buf[e, pl.ds(k * 16, 16)] = (
                            buf[e, pl.ds(k * 16, 16)] * wsp)

        def gather(j, b):
            return pltpu.async_copy(
                table_h.at[src_v.at[pl.ds(j * CHUNK, CHUNK)]],
                rows[b], gsem[b])

        def gather_wait(b):
            pltpu.make_async_copy(
                table_h.at[src_v.at[pl.ds(0, CHUNK)]],
                rows[b], gsem[b]).wait()

        def scatter(j, b):
            return pltpu.async_copy(
                rows[b], agg_sh.at[dst_v.at[j]], ssem[b], add=True)

        def scatter_wait(b):
            pltpu.make_async_copy(
                rows[b], agg_sh.at[dst_v.at[0]], ssem[b]).wait()

        def group_body(g, _):
            pltpu.async_copy(src_h.at[pl.ds(ebase + g * GE, GE)], src_v, tsem)
            pltpu.async_copy(dst_h.at[pl.ds(ebase + g * GE, GE)], dst1_v, tsem)
            pltpu.async_copy(w_h.at[pl.ds(ebase + g * GE, GE)], w_v, tsem)
            pltpu.make_async_copy(
                src_h.at[pl.ds(ebase + g * GE, GE)], src_v, tsem).wait()
            pltpu.make_async_copy(
                dst_h.at[pl.ds(ebase + g * GE, GE)], dst1_v, tsem).wait()
            pltpu.make_async_copy(
                w_h.at[pl.ds(ebase + g * GE, GE)], w_v, tsem).wait()

            @plsc.parallel_loop(0, G)
            def drow(j):
                for q in range(CHUNK // 16):
                    dst_v[j, pl.ds(q * 16, 16)] = (
                        dst1_v[pl.ds(j * CHUNK + q * 16, 16)])

            if off is not None:
                @plsc.parallel_loop(0, GE // 16)
                def offs(t):
                    src_v[pl.ds(t * 16, 16)] = (
                        src_v[pl.ds(t * 16, 16)] + off)

            g0 = gather(0, 0)
            gather(1, 1)
            g0.wait()
            scale(rows[0], 0)
            scatter(0, 0)
            gather(2, 2)

            def triple(t, _):
                c1 = 3 * t + 1
                gather_wait(1)
                scale(rows[1], c1)
                s1 = scatter(c1, 1)
                scatter_wait(0)
                gather(c1 + 2, 0)

                gather_wait(2)
                scale(rows[2], c1 + 1)
                s2 = scatter(c1 + 1, 2)
                s1.wait()

                @pl.when(t < NBODY - 1)
                def _():
                    gather(c1 + 3, 1)

                gather_wait(0)
                scale(rows[0], c1 + 2)
                scatter(c1 + 2, 0)
                s2.wait()

                @pl.when(t < NBODY - 1)
                def _():
                    gather(c1 + 4, 2)
                return 0

            lax.fori_loop(0, NBODY, triple, 0)
            scatter_wait(0)
            return 0

        lax.fori_loop(0, ngroups, group_body, 0)
        plsc.subcore_barrier()

        pltpu.sync_copy(
            agg_sh.at[pl.ds(s * rows_per_tile, rows_per_tile)],
            out_h.at[c, pl.ds(s * rows_per_tile, rows_per_tile)])

    return seg


def _l1_body(aggp_ref, x_ref, wr_ref, wro_ref, b_ref, a_ref, out_ref):
    agg = aggp_ref[0] + aggp_ref[1]
    t = jnp.dot(agg, wr_ref[...], preferred_element_type=jnp.float32)
    t += jnp.dot(x_ref[...], wro_ref[...], preferred_element_type=jnp.float32)
    t += b_ref[...]
    h = jnp.where(t >= 0.0, t, a_ref[...] * t)
    out_ref[0] = h[:, :D]
    out_ref[1] = h[:, D:]


def _l2_body(agg_ref, h1_ref, wr_ref, wro_ref, b_ref, a_ref, wfc_ref,
             bfc_ref, out_ref):
    t = jnp.dot(agg_ref[0], wr_ref[:D], preferred_element_type=jnp.float32)
    t += jnp.dot(agg_ref[1], wr_ref[D:], preferred_element_type=jnp.float32)
    t += jnp.dot(h1_ref[0], wro_ref[:D], preferred_element_type=jnp.float32)
    t += jnp.dot(h1_ref[1], wro_ref[D:], preferred_element_type=jnp.float32)
    t += b_ref[...]
    h = jnp.where(t >= 0.0, t, a_ref[...] * t)
    out_ref[...] = (
        jnp.dot(h, wfc_ref[...], preferred_element_type=jnp.float32)
        + bfc_ref[...])


_RB = 1000


def _tc_layer1(aggp, x, W_rel1, b_rel1, W_root1, a1):
    grid = (N // _RB,)
    return pl.pallas_call(
        _l1_body,
        grid=grid,
        in_specs=[
            pl.BlockSpec((NC, _RB, D), lambda i: (0, i, 0)),
            pl.BlockSpec((_RB, D), lambda i: (i, 0)),
            pl.BlockSpec((D, 2 * D), lambda i: (0, 0)),
            pl.BlockSpec((D, 2 * D), lambda i: (0, 0)),
            pl.BlockSpec((1, 2 * D), lambda i: (0, 0)),
            pl.BlockSpec((1, 1), lambda i: (0, 0)),
        ],
        out_specs=pl.BlockSpec((NC, _RB, D), lambda i: (0, i, 0)),
        out_shape=jax.ShapeDtypeStruct((NC, N, D), jnp.float32),
    )(aggp, x, W_rel1, W_root1, b_rel1.reshape(1, -1), a1.reshape(1, 1))


def _tc_layer2(agg2, h1s, W_rel2, b_rel2, W_root2, a2, W_fc, b_fc):
    grid = (N // _RB,)
    return pl.pallas_call(
        _l2_body,
        grid=grid,
        in_specs=[
            pl.BlockSpec((NC, _RB, D), lambda i: (0, i, 0)),
            pl.BlockSpec((NC, _RB, D), lambda i: (0, i, 0)),
            pl.BlockSpec((2 * D, 2 * D), lambda i: (0, 0)),
            pl.BlockSpec((2 * D, 2 * D), lambda i: (0, 0)),
            pl.BlockSpec((1, 2 * D), lambda i: (0, 0)),
            pl.BlockSpec((1, 1), lambda i: (0, 0)),
            pl.BlockSpec((2 * D, D), lambda i: (0, 0)),
            pl.BlockSpec((1, D), lambda i: (0, 0)),
        ],
        out_specs=pl.BlockSpec((_RB, D), lambda i: (i, 0)),
        out_shape=jax.ShapeDtypeStruct((N, D), jnp.float32),
    )(agg2, h1s, W_rel2, W_root2, b_rel2.reshape(1, -1), a2.reshape(1, 1),
      W_fc, b_fc.reshape(1, -1))


def kernel(x, edge_index, edge_attr, W_rel1, b_rel1, W_root1, a1,
           W_rel2, b_rel2, W_root2, a2, W_fc, b_fc):
    E = edge_index.shape[1]
    src = edge_index[0].astype(jnp.int32)
    dst = edge_index[1].astype(jnp.int32)
    w = edge_attr.astype(jnp.float32)

    seg1 = _make_segsum(E, mode=1)
    seg2 = _make_segsum(E, mode=2)

    agg1p = seg1(x, src, dst, w)
    h1s = _tc_layer1(agg1p, x, W_rel1, b_rel1, W_root1, a1)
    agg2 = seg2(h1s.reshape(NC * N, D), src, dst, w)
    out = _tc_layer2(agg2, h1s, W_rel2, b_rel2, W_root2, a2, W_fc, b_fc)
    return out.reshape(BATCH, -1, D)
